# baseline (speedup 1.0000x reference)
import functools

import jax
import jax.numpy as jnp
import numpy as np
from jax.experimental import pallas as pl
from jax.experimental.pallas import tpu as pltpu

G, A, N, D, L, NL, MAXA, NF = 625, 16, 10000, 128, 256, 4, 100, 10
GB = 125
NB = G // GB
NPB = GB * A
EPB = GB * A * A
EP2 = EPB // 2
HA = A // 2
D2 = 2 * D
_TWO_PI = float(2.0 * np.pi)


def _angle_matrix():
    m = np.zeros((16, 128), np.float32)
    for half in range(2):
        r0, c0 = 8 * half, 64 * half
        for c in range(3):
            for f in range(NF):
                m[r0 + c, c0 + c * NF + f] = float(f)
                m[r0 + c, c0 + 30 + c * NF + f] = float(f)
        m[r0 + 3, c0 + 30:c0 + 60] = 0.25
    return m


_SIN_C = (6.283182792843449, -41.34141933301581, 81.5961374087892,
          -76.57967400035034, 41.203682075143085, -12.268761447387364)


def _sin2pi(m):
    r = m - jnp.round(m)
    t = r * r
    p = jnp.float32(_SIN_C[5])
    for c in _SIN_C[4::-1]:
        p = p * t + jnp.float32(c)
    return r * p


def _silu(x):
    u = 0.5 * x
    return u + u * jnp.tanh(u)


def _fused_kernel(at_ref, t_ref, frac_ref, lat_ref,
                  m2_ref,
                  emb_ref, wla_ref, wlb_ref, bl_ref,
                  whi_ref, whj_ref, wlat_ref, wfd_ref, eb1_ref,
                  we2_ref, eb2_ref,
                  wn1a_ref, wn1b_ref, nb1_ref, wn2_ref, nb2_ref,
                  wc_ref, wl_ref,
                  coord_ref, latout_ref):
    f32 = jnp.float32
    dot = functools.partial(jnp.dot, preferred_element_type=f32)

    at = at_ref[0]
    lane = jax.lax.broadcasted_iota(jnp.int32, (NPB, D), 1).astype(f32)
    onehot = (lane == at).astype(f32)
    emb = dot(onehot, emb_ref[...])
    tb = t_ref[0]
    t_pa = jnp.broadcast_to(tb[:, None, :], (GB, A, L)).reshape(NPB, L)
    nf = emb @ wla_ref[...] + t_pa @ wlb_ref[...] + bl_ref[...]

    f3 = frac_ref[0]
    f2 = f3.reshape(NPB, 3)
    f6 = jnp.concatenate([f3[:, :HA, :], f3[:, HA:, :]], axis=2)
    fdst = jnp.concatenate([f6] * A, axis=1).reshape(EP2, 6)
    fsrc = jnp.broadcast_to(
        jnp.concatenate([f2, f2], axis=1)[:, None, :], (NPB, HA, 6)
    ).reshape(EP2, 6)
    fd = fdst - fsrc
    fd = fd - jnp.floor(fd)
    ones = jnp.ones((EP2, 1), f32)
    zero4 = jnp.zeros((EP2, 4), f32)
    fdh = jnp.concatenate(
        [fd[:, :3], ones, zero4, fd[:, 3:], ones, zero4], axis=1)
    fe = _sin2pi(dot(fdh, m2_ref[...]))

    lat9 = lat_ref[0]
    ip_cols = []
    for i in range(3):
        for j in range(3):
            s = (lat9[:, 3 * i + 0:3 * i + 1] * lat9[:, 3 * j + 0:3 * j + 1]
                 + lat9[:, 3 * i + 1:3 * i + 2] * lat9[:, 3 * j + 1:3 * j + 2]
                 + lat9[:, 3 * i + 2:3 * i + 3] * lat9[:, 3 * j + 2:3 * j + 3])
            ip_cols.append(s)
    latip = jnp.concatenate(ip_cols + [jnp.zeros((GB, 7), f32)], axis=1)

    inv_a = f32(1.0 / A)
    for l in range(NL):
        le = dot(latip, wlat_ref[l])
        le_n = jnp.broadcast_to(le[:, None, :], (GB, A, D)).reshape(NPB, D)
        pi = dot(nf, whi_ref[l]) + le_n + eb1_ref[l]
        pj = dot(nf, whj_ref[l])
        fdw = jnp.concatenate(
            [dot(fe[:, :64], wfd_ref[l]), dot(fe[:, 64:], wfd_ref[l])], axis=1)
        hi_e = jnp.broadcast_to(
            jnp.concatenate([pi, pi], axis=1)[:, None, :], (NPB, HA, D2)
        ).reshape(EP2, D2)
        pj3 = pj.reshape(GB, A, D)
        pjp = jnp.concatenate([pj3[:, :HA, :], pj3[:, HA:, :]], axis=2)
        hj_e = jnp.concatenate([pjp] * A, axis=1).reshape(EP2, D2)
        h = _silu(hi_e + hj_e + fdw)
        ef = jnp.concatenate(
            [dot(h[:, :D], we2_ref[l]), dot(h[:, D:], we2_ref[l])], axis=1)
        ef = _silu(ef + eb2_ref[l])
        agg = (ef[:, :D] + ef[:, D:]).reshape(NPB, HA, D).sum(axis=1) * inv_a
        h2 = _silu(dot(nf, wn1a_ref[l]) + dot(agg, wn1b_ref[l]) + nb1_ref[l])
        nf = nf + _silu(dot(h2, wn2_ref[l]) + nb2_ref[l])

    co = dot(nf, wc_ref[...])
    coord_ref[0] = co[:, :3]
    gf = nf.reshape(GB, A, D).sum(axis=1) * inv_a
    l9 = dot(gf, wl_ref[...])
    out_cols = []
    for i in range(3):
        for k in range(3):
            s = (l9[:, 3 * i + 0:3 * i + 1] * lat9[:, 0 + k:1 + k]
                 + l9[:, 3 * i + 1:3 * i + 2] * lat9[:, 3 + k:4 + k]
                 + l9[:, 3 * i + 2:3 * i + 3] * lat9[:, 6 + k:7 + k])
            out_cols.append(s)
    latout_ref[0] = jnp.concatenate(out_cols, axis=1)


def kernel(t, atom_types, frac_coords, lattices, num_atoms, node2graph,
           emb_table, W_latent, b_latent, edge_w1, edge_b1, edge_w2, edge_b2,
           node_w1, node_b1, node_w2, node_b2, W_coord, W_lattice):
    f32 = jnp.float32
    at_f = (atom_types.astype(f32) - 1.0).reshape(NB, NPB, 1)
    t_r = t.reshape(NB, GB, L)
    frac_r = frac_coords.reshape(NB, GB, A, 3)
    lat_r = lattices.reshape(NB, GB, 9)
    m2 = jnp.asarray(_angle_matrix())

    emb_pad = jnp.zeros((D, D), f32).at[:MAXA].set(emb_table)
    wla = W_latent[:, :D].T
    wlb = W_latent[:, D:].T
    bl = b_latent.reshape(1, D)
    e1t = jnp.swapaxes(edge_w1, 1, 2)
    whi = e1t[:, :D]
    whj = e1t[:, D:2 * D]
    wlat = jnp.zeros((NL, 16, D), f32).at[:, :9].set(e1t[:, 2 * D:2 * D + 9])
    wfd = jnp.zeros((NL, 64, D), f32).at[:, :60].set(e1t[:, 2 * D + 9:])
    eb1 = edge_b1.reshape(NL, 1, D)
    we2 = jnp.swapaxes(edge_w2, 1, 2)
    eb2 = jnp.tile(edge_b2.reshape(NL, 1, D), (1, 1, 2))
    n1t = jnp.swapaxes(node_w1, 1, 2)
    wn1a = n1t[:, :D]
    wn1b = n1t[:, D:]
    nb1 = node_b1.reshape(NL, 1, D)
    wn2 = jnp.swapaxes(node_w2, 1, 2)
    nb2 = node_b2.reshape(NL, 1, D)
    wc = jnp.zeros((D, 8), f32).at[:, :3].set(W_coord.T)
    wl = jnp.zeros((D, 16), f32).at[:, :9].set(W_lattice.T)

    def blk(shape):
        return pl.BlockSpec(shape, lambda i: (i,) + (0,) * (len(shape) - 1))

    def const(shape):
        return pl.BlockSpec(shape, lambda i: (0,) * len(shape))

    coord_r, latout_r = pl.pallas_call(
        _fused_kernel,
        grid=(NB,),
        in_specs=[
            blk((1, NPB, 1)), blk((1, GB, L)), blk((1, GB, A, 3)), blk((1, GB, 9)),
            const((16, D)),
            const((D, D)), const((D, D)), const((L, D)), const((1, D)),
            const((NL, D, D)), const((NL, D, D)), const((NL, 16, D)),
            const((NL, 64, D)), const((NL, 1, D)),
            const((NL, D, D)), const((NL, 1, D2)),
            const((NL, D, D)), const((NL, D, D)), const((NL, 1, D)),
            const((NL, D, D)), const((NL, 1, D)),
            const((D, 8)), const((D, 16)),
        ],
        out_specs=[blk((1, NPB, 3)), blk((1, GB, 9))],
        out_shape=[
            jax.ShapeDtypeStruct((NB, NPB, 3), f32),
            jax.ShapeDtypeStruct((NB, GB, 9), f32),
        ],
        compiler_params=pltpu.CompilerParams(
            dimension_semantics=("parallel",),
        ),
    )(at_f, t_r, frac_r, lat_r, m2, emb_pad, wla, wlb, bl, whi, whj, wlat,
      wfd, eb1, we2, eb2, wn1a, wn1b, nb1, wn2, nb2, wc, wl)

    coord_out = coord_r.reshape(N, 3)
    lattice_out = latout_r.reshape(G, 3, 3)
    return lattice_out, coord_out


# speedup vs baseline: 1.1208x; 1.1208x over previous
import functools

import jax
import jax.numpy as jnp
import numpy as np
from jax.experimental import pallas as pl
from jax.experimental.pallas import tpu as pltpu

G, A, N, D, L, NL, MAXA, NF = 625, 16, 10000, 128, 256, 4, 100, 10
GB = 25
NB = G // GB
NPB = GB * A
EPB = GB * A * A
EP2 = EPB // 2
HA = A // 2
D2 = 2 * D
_TWO_PI = float(2.0 * np.pi)


def _angle_matrix():
    m = np.zeros((16, 128), np.float32)
    for half in range(2):
        r0, c0 = 8 * half, 64 * half
        for c in range(3):
            for f in range(NF):
                m[r0 + c, c0 + c * NF + f] = float(f)
                m[r0 + c, c0 + 30 + c * NF + f] = float(f)
        m[r0 + 3, c0 + 30:c0 + 60] = 0.25
    return m


_SIN_C = (6.283182792843449, -41.34141933301581, 81.5961374087892,
          -76.57967400035034, 41.203682075143085, -12.268761447387364)


def _sin2pi(m):
    r = m - jnp.round(m)
    t = r * r
    p = jnp.float32(_SIN_C[5])
    for c in _SIN_C[4::-1]:
        p = p * t + jnp.float32(c)
    return r * p


def _silu(x):
    u = 0.5 * x
    return u + u * jnp.tanh(u)


def _fused_kernel(at_ref, t_ref, frac_ref, lat_ref,
                  m2_ref,
                  emb_ref, wlb_ref, bl_ref,
                  whi_ref, whj_ref, wlat_ref, wfd_ref, eb1_ref,
                  we2_ref, eb2_ref,
                  wn1a_ref, wn1b_ref, nb1_ref, wn2_ref, nb2_ref,
                  wc_ref, wl_ref,
                  coord_ref, latout_ref):
    f32 = jnp.float32
    dot = functools.partial(jnp.dot, preferred_element_type=f32)

    at = at_ref[0]
    lane = jax.lax.broadcasted_iota(jnp.int32, (NPB, D), 1).astype(f32)
    onehot = (lane == at).astype(f32)
    tb = t_ref[0]
    tw = dot(tb, wlb_ref[...]) + bl_ref[...]
    tw_n = jnp.broadcast_to(tw[:, None, :], (GB, A, D)).reshape(NPB, D)
    nf = dot(onehot, emb_ref[...]) + tw_n

    f3 = frac_ref[0]
    f2 = f3.reshape(NPB, 3)
    f6 = jnp.concatenate([f3[:, :HA, :], f3[:, HA:, :]], axis=2)
    fdst = jnp.concatenate([f6] * A, axis=1).reshape(EP2, 6)
    fsrc = jnp.broadcast_to(
        jnp.concatenate([f2, f2], axis=1)[:, None, :], (NPB, HA, 6)
    ).reshape(EP2, 6)
    fd = fdst - fsrc
    fd = fd - jnp.floor(fd)
    ones = jnp.ones((EP2, 1), f32)
    zero4 = jnp.zeros((EP2, 4), f32)
    fdh = jnp.concatenate(
        [fd[:, :3], ones, zero4, fd[:, 3:], ones, zero4], axis=1)
    fe = _sin2pi(dot(fdh, m2_ref[...]))

    lat9 = lat_ref[0]
    ip_cols = []
    for i in range(3):
        for j in range(3):
            s = (lat9[:, 3 * i + 0:3 * i + 1] * lat9[:, 3 * j + 0:3 * j + 1]
                 + lat9[:, 3 * i + 1:3 * i + 2] * lat9[:, 3 * j + 1:3 * j + 2]
                 + lat9[:, 3 * i + 2:3 * i + 3] * lat9[:, 3 * j + 2:3 * j + 3])
            ip_cols.append(s)
    latip = jnp.concatenate(ip_cols + [jnp.zeros((GB, 7), f32)], axis=1)

    inv_a = f32(1.0 / A)
    for l in range(NL):
        le = dot(latip, wlat_ref[l])
        le_n = jnp.broadcast_to(le[:, None, :], (GB, A, D)).reshape(NPB, D)
        pi = dot(nf, whi_ref[l]) + le_n + eb1_ref[l]
        pj = dot(nf, whj_ref[l])
        fdw = jnp.concatenate(
            [dot(fe[:, :64], wfd_ref[l]), dot(fe[:, 64:], wfd_ref[l])], axis=1)
        hi_e = jnp.broadcast_to(
            jnp.concatenate([pi, pi], axis=1)[:, None, :], (NPB, HA, D2)
        ).reshape(EP2, D2)
        pj3 = pj.reshape(GB, A, D)
        pjp = jnp.concatenate([pj3[:, :HA, :], pj3[:, HA:, :]], axis=2)
        hj_e = jnp.concatenate([pjp] * A, axis=1).reshape(EP2, D2)
        h = _silu(hi_e + hj_e + fdw)
        ef = jnp.concatenate(
            [dot(h[:, :D], we2_ref[l]), dot(h[:, D:], we2_ref[l])], axis=1)
        ef = _silu(ef + eb2_ref[l])
        agg = (ef[:, :D] + ef[:, D:]).reshape(NPB, HA, D).sum(axis=1) * inv_a
        h2 = _silu(dot(nf, wn1a_ref[l]) + dot(agg, wn1b_ref[l]) + nb1_ref[l])
        nf = nf + _silu(dot(h2, wn2_ref[l]) + nb2_ref[l])

    co = dot(nf, wc_ref[...])
    coord_ref[0] = co[:, :3]
    gf = nf.reshape(GB, A, D).sum(axis=1) * inv_a
    l9 = dot(gf, wl_ref[...])
    out_cols = []
    for i in range(3):
        for k in range(3):
            s = (l9[:, 3 * i + 0:3 * i + 1] * lat9[:, 0 + k:1 + k]
                 + l9[:, 3 * i + 1:3 * i + 2] * lat9[:, 3 + k:4 + k]
                 + l9[:, 3 * i + 2:3 * i + 3] * lat9[:, 6 + k:7 + k])
            out_cols.append(s)
    latout_ref[0] = jnp.concatenate(out_cols, axis=1)


def kernel(t, atom_types, frac_coords, lattices, num_atoms, node2graph,
           emb_table, W_latent, b_latent, edge_w1, edge_b1, edge_w2, edge_b2,
           node_w1, node_b1, node_w2, node_b2, W_coord, W_lattice):
    f32 = jnp.float32
    at_f = (atom_types.astype(f32) - 1.0).reshape(NB, NPB, 1)
    t_r = t.reshape(NB, GB, L)
    frac_r = frac_coords.reshape(NB, GB, A, 3)
    lat_r = lattices.reshape(NB, GB, 9)
    m2 = jnp.asarray(_angle_matrix())

    wla = W_latent[:, :D].T
    emb_pad = jnp.zeros((D, D), f32).at[:MAXA].set(emb_table) @ wla
    wlb = W_latent[:, D:].T
    bl = b_latent.reshape(1, D)
    e1t = jnp.swapaxes(edge_w1, 1, 2)
    whi = e1t[:, :D]
    whj = e1t[:, D:2 * D]
    wlat = jnp.zeros((NL, 16, D), f32).at[:, :9].set(e1t[:, 2 * D:2 * D + 9])
    wfd = jnp.zeros((NL, 64, D), f32).at[:, :60].set(e1t[:, 2 * D + 9:])
    eb1 = edge_b1.reshape(NL, 1, D)
    we2 = jnp.swapaxes(edge_w2, 1, 2)
    eb2 = jnp.tile(edge_b2.reshape(NL, 1, D), (1, 1, 2))
    n1t = jnp.swapaxes(node_w1, 1, 2)
    wn1a = n1t[:, :D]
    wn1b = n1t[:, D:]
    nb1 = node_b1.reshape(NL, 1, D)
    wn2 = jnp.swapaxes(node_w2, 1, 2)
    nb2 = node_b2.reshape(NL, 1, D)
    wc = jnp.zeros((D, 8), f32).at[:, :3].set(W_coord.T)
    wl = jnp.zeros((D, 16), f32).at[:, :9].set(W_lattice.T)

    def blk(shape):
        return pl.BlockSpec(shape, lambda i: (i,) + (0,) * (len(shape) - 1))

    def const(shape):
        return pl.BlockSpec(shape, lambda i: (0,) * len(shape))

    coord_r, latout_r = pl.pallas_call(
        _fused_kernel,
        grid=(NB,),
        in_specs=[
            blk((1, NPB, 1)), blk((1, GB, L)), blk((1, GB, A, 3)), blk((1, GB, 9)),
            const((16, D)),
            const((D, D)), const((L, D)), const((1, D)),
            const((NL, D, D)), const((NL, D, D)), const((NL, 16, D)),
            const((NL, 64, D)), const((NL, 1, D)),
            const((NL, D, D)), const((NL, 1, D2)),
            const((NL, D, D)), const((NL, D, D)), const((NL, 1, D)),
            const((NL, D, D)), const((NL, 1, D)),
            const((D, 8)), const((D, 16)),
        ],
        out_specs=[blk((1, NPB, 3)), blk((1, GB, 9))],
        out_shape=[
            jax.ShapeDtypeStruct((NB, NPB, 3), f32),
            jax.ShapeDtypeStruct((NB, GB, 9), f32),
        ],
        compiler_params=pltpu.CompilerParams(
            dimension_semantics=("parallel",),
        ),
    )(at_f, t_r, frac_r, lat_r, m2, emb_pad, wlb, bl, whi, whj, wlat,
      wfd, eb1, we2, eb2, wn1a, wn1b, nb1, wn2, nb2, wc, wl)

    coord_out = coord_r.reshape(N, 3)
    lattice_out = latout_r.reshape(G, 3, 3)
    return lattice_out, coord_out


# speedup vs baseline: 1.3250x; 1.1822x over previous
import functools

import jax
import jax.numpy as jnp
import numpy as np
from jax.experimental import pallas as pl
from jax.experimental.pallas import tpu as pltpu

G, A, N, D, L, NL, MAXA, NF = 625, 16, 10000, 128, 256, 4, 100, 10
GB = 25
NB = G // GB
NPB = GB * A
EPB = GB * A * A
EP2 = EPB // 2
HA = A // 2
D2 = 2 * D
_TWO_PI = float(2.0 * np.pi)


def _angle_matrix():
    m = np.zeros((16, 128), np.float32)
    for half in range(2):
        r0, c0 = 8 * half, 64 * half
        for c in range(3):
            for f in range(NF):
                m[r0 + c, c0 + c * NF + f] = float(f)
                m[r0 + c, c0 + 30 + c * NF + f] = float(f)
        m[r0 + 3, c0 + 30:c0 + 60] = 0.25
    return m


_SIN_C = (6.283182792843449, -41.34141933301581, 81.5961374087892,
          -76.57967400035034, 41.203682075143085, -12.268761447387364)


def _sin2pi(m):
    r = m - jnp.round(m)
    t = r * r
    p = jnp.float32(_SIN_C[5])
    for c in _SIN_C[4::-1]:
        p = p * t + jnp.float32(c)
    return r * p


def _silu(x):
    u = 0.5 * x
    return u + u * jnp.tanh(u)


def _fused_kernel(at_ref, t_ref, frac_ref, lat_ref,
                  m2_ref,
                  emb_ref, wlb_ref, bl_ref,
                  whi_ref, whj_ref, wlat_ref, wfd_ref, eb1_ref,
                  we2_ref, eb2_ref,
                  wn1a_ref, wn1b_ref, nb1_ref, wn2_ref, nb2_ref,
                  wc_ref, wl_ref,
                  coord_ref, latout_ref):
    f32 = jnp.float32
    dot = functools.partial(jnp.dot, preferred_element_type=f32)

    at = at_ref[0]
    lane = jax.lax.broadcasted_iota(jnp.int32, (NPB, D), 1).astype(f32)
    onehot = (lane == at).astype(f32)
    tb = t_ref[0]
    tw = dot(tb, wlb_ref[...]) + bl_ref[...]
    tw_n = jnp.broadcast_to(tw[:, None, :], (GB, A, D)).reshape(NPB, D)
    nf = dot(onehot, emb_ref[...]) + tw_n

    f3 = frac_ref[0]
    f2 = f3.reshape(NPB, 3)
    f6 = jnp.concatenate([f3[:, :HA, :], f3[:, HA:, :]], axis=2)
    fdst = jnp.broadcast_to(f6[:, :, None, :], (GB, HA, A, 6)).reshape(EP2, 6)
    f2c = jnp.concatenate([f2, f2], axis=1)
    fsrc = jnp.broadcast_to(
        f2c.reshape(GB, 1, A, 6), (GB, HA, A, 6)).reshape(EP2, 6)
    fd = fdst - fsrc
    fd = fd - jnp.floor(fd)
    ones = jnp.ones((EP2, 1), f32)
    zero4 = jnp.zeros((EP2, 4), f32)
    fdh = jnp.concatenate(
        [fd[:, :3], ones, zero4, fd[:, 3:], ones, zero4], axis=1)
    fe = _sin2pi(dot(fdh, m2_ref[...]))

    lat9 = lat_ref[0]
    ip_cols = []
    for i in range(3):
        for j in range(3):
            s = (lat9[:, 3 * i + 0:3 * i + 1] * lat9[:, 3 * j + 0:3 * j + 1]
                 + lat9[:, 3 * i + 1:3 * i + 2] * lat9[:, 3 * j + 1:3 * j + 2]
                 + lat9[:, 3 * i + 2:3 * i + 3] * lat9[:, 3 * j + 2:3 * j + 3])
            ip_cols.append(s)
    latip = jnp.concatenate(ip_cols + [jnp.zeros((GB, 7), f32)], axis=1)

    inv_a = f32(1.0 / A)
    for l in range(NL):
        le = dot(latip, wlat_ref[l])
        le_n = jnp.broadcast_to(le[:, None, :], (GB, A, D)).reshape(NPB, D)
        pi = dot(nf, whi_ref[l]) + le_n + eb1_ref[l]
        pj = dot(nf, whj_ref[l])
        fdw = jnp.concatenate(
            [dot(fe[:, :64], wfd_ref[l]), dot(fe[:, 64:], wfd_ref[l])], axis=1)
        pi2 = jnp.concatenate([pi, pi], axis=1)
        hi_e = jnp.broadcast_to(
            pi2.reshape(GB, 1, A, D2), (GB, HA, A, D2)).reshape(EP2, D2)
        pj3 = pj.reshape(GB, A, D)
        pjp = jnp.concatenate([pj3[:, :HA, :], pj3[:, HA:, :]], axis=2)
        hj_e = jnp.broadcast_to(
            pjp[:, :, None, :], (GB, HA, A, D2)).reshape(EP2, D2)
        h = _silu(hi_e + hj_e + fdw)
        ef = jnp.concatenate(
            [dot(h[:, :D], we2_ref[l]), dot(h[:, D:], we2_ref[l])], axis=1)
        ef = _silu(ef + eb2_ref[l])
        agg = ((ef[:, :D] + ef[:, D:]).reshape(GB, HA, A, D)
               .sum(axis=1).reshape(NPB, D) * inv_a)
        h2 = _silu(dot(nf, wn1a_ref[l]) + dot(agg, wn1b_ref[l]) + nb1_ref[l])
        nf = nf + _silu(dot(h2, wn2_ref[l]) + nb2_ref[l])

    co = dot(nf, wc_ref[...])
    coord_ref[0] = co[:, :3]
    gf = nf.reshape(GB, A, D).sum(axis=1) * inv_a
    l9 = dot(gf, wl_ref[...])
    out_cols = []
    for i in range(3):
        for k in range(3):
            s = (l9[:, 3 * i + 0:3 * i + 1] * lat9[:, 0 + k:1 + k]
                 + l9[:, 3 * i + 1:3 * i + 2] * lat9[:, 3 + k:4 + k]
                 + l9[:, 3 * i + 2:3 * i + 3] * lat9[:, 6 + k:7 + k])
            out_cols.append(s)
    latout_ref[0] = jnp.concatenate(out_cols, axis=1)


def kernel(t, atom_types, frac_coords, lattices, num_atoms, node2graph,
           emb_table, W_latent, b_latent, edge_w1, edge_b1, edge_w2, edge_b2,
           node_w1, node_b1, node_w2, node_b2, W_coord, W_lattice):
    f32 = jnp.float32
    at_f = (atom_types.astype(f32) - 1.0).reshape(NB, NPB, 1)
    t_r = t.reshape(NB, GB, L)
    frac_r = frac_coords.reshape(NB, GB, A, 3)
    lat_r = lattices.reshape(NB, GB, 9)
    m2 = jnp.asarray(_angle_matrix())

    wla = W_latent[:, :D].T
    emb_pad = jnp.zeros((D, D), f32).at[:MAXA].set(emb_table) @ wla
    wlb = W_latent[:, D:].T
    bl = b_latent.reshape(1, D)
    e1t = jnp.swapaxes(edge_w1, 1, 2)
    whi = e1t[:, :D]
    whj = e1t[:, D:2 * D]
    wlat = jnp.zeros((NL, 16, D), f32).at[:, :9].set(e1t[:, 2 * D:2 * D + 9])
    wfd = jnp.zeros((NL, 64, D), f32).at[:, :60].set(e1t[:, 2 * D + 9:])
    eb1 = edge_b1.reshape(NL, 1, D)
    we2 = jnp.swapaxes(edge_w2, 1, 2)
    eb2 = jnp.tile(edge_b2.reshape(NL, 1, D), (1, 1, 2))
    n1t = jnp.swapaxes(node_w1, 1, 2)
    wn1a = n1t[:, :D]
    wn1b = n1t[:, D:]
    nb1 = node_b1.reshape(NL, 1, D)
    wn2 = jnp.swapaxes(node_w2, 1, 2)
    nb2 = node_b2.reshape(NL, 1, D)
    wc = jnp.zeros((D, 8), f32).at[:, :3].set(W_coord.T)
    wl = jnp.zeros((D, 16), f32).at[:, :9].set(W_lattice.T)

    def blk(shape):
        return pl.BlockSpec(shape, lambda i: (i,) + (0,) * (len(shape) - 1))

    def const(shape):
        return pl.BlockSpec(shape, lambda i: (0,) * len(shape))

    coord_r, latout_r = pl.pallas_call(
        _fused_kernel,
        grid=(NB,),
        in_specs=[
            blk((1, NPB, 1)), blk((1, GB, L)), blk((1, GB, A, 3)), blk((1, GB, 9)),
            const((16, D)),
            const((D, D)), const((L, D)), const((1, D)),
            const((NL, D, D)), const((NL, D, D)), const((NL, 16, D)),
            const((NL, 64, D)), const((NL, 1, D)),
            const((NL, D, D)), const((NL, 1, D2)),
            const((NL, D, D)), const((NL, D, D)), const((NL, 1, D)),
            const((NL, D, D)), const((NL, 1, D)),
            const((D, 8)), const((D, 16)),
        ],
        out_specs=[blk((1, NPB, 3)), blk((1, GB, 9))],
        out_shape=[
            jax.ShapeDtypeStruct((NB, NPB, 3), f32),
            jax.ShapeDtypeStruct((NB, GB, 9), f32),
        ],
        compiler_params=pltpu.CompilerParams(
            dimension_semantics=("parallel",),
        ),
    )(at_f, t_r, frac_r, lat_r, m2, emb_pad, wlb, bl, whi, whj, wlat,
      wfd, eb1, we2, eb2, wn1a, wn1b, nb1, wn2, nb2, wc, wl)

    coord_out = coord_r.reshape(N, 3)
    lattice_out = latout_r.reshape(G, 3, 3)
    return lattice_out, coord_out


# speedup vs baseline: 1.3922x; 1.0507x over previous
import functools

import jax
import jax.numpy as jnp
import numpy as np
from jax.experimental import pallas as pl
from jax.experimental.pallas import tpu as pltpu

G, A, N, D, L, NL, MAXA, NF = 625, 16, 10000, 128, 256, 4, 100, 10
GB = 25
NB = G // GB
NPB = GB * A
EPB = GB * A * A
EP2 = EPB // 2
HA = A // 2
D2 = 2 * D
_TWO_PI = float(2.0 * np.pi)


def _angle_matrix():
    m = np.zeros((16, 128), np.float32)
    for half in range(2):
        r0, c0 = 8 * half, 64 * half
        for c in range(3):
            for f in range(NF):
                m[r0 + c, c0 + c * NF + f] = float(f)
                m[r0 + c, c0 + 30 + c * NF + f] = float(f)
        m[r0 + 3, c0 + 30:c0 + 60] = 0.25
    return m


_SIN_C = (6.283182792843449, -41.34141933301581, 81.5961374087892,
          -76.57967400035034, 41.203682075143085, -12.268761447387364)


def _sin2pi(m):
    r = m - jnp.round(m)
    t = r * r
    p = jnp.float32(_SIN_C[5])
    for c in _SIN_C[4::-1]:
        p = p * t + jnp.float32(c)
    return r * p


def _silu2(u):
    return u + u * jnp.tanh(u)


def _fused_kernel(at_ref, t_ref, frac_ref, lat_ref,
                  m2_ref,
                  emb_ref, wlb_ref, bl_ref,
                  whi_ref, whj_ref, wlat_ref, wfd_ref, eb1_ref,
                  we2_ref, eb2_ref,
                  wn1a_ref, wn1b_ref, nb1_ref, wn2_ref, nb2_ref,
                  wc_ref, wl_ref,
                  coord_ref, latout_ref):
    f32 = jnp.float32
    dot = functools.partial(jnp.dot, preferred_element_type=f32)

    at = at_ref[0]
    lane = jax.lax.broadcasted_iota(jnp.int32, (NPB, D), 1).astype(f32)
    onehot = (lane == at).astype(f32)
    tb = t_ref[0]
    tw = dot(tb, wlb_ref[...]) + bl_ref[...]
    tw_n = jnp.broadcast_to(tw[:, None, :], (GB, A, D)).reshape(NPB, D)
    nf = dot(onehot, emb_ref[...]) + tw_n

    f3 = frac_ref[0]
    f2 = f3.reshape(NPB, 3)
    f6 = jnp.concatenate([f3[:, :HA, :], f3[:, HA:, :]], axis=2)
    fdst = jnp.broadcast_to(f6[:, :, None, :], (GB, HA, A, 6)).reshape(EP2, 6)
    f2c = jnp.concatenate([f2, f2], axis=1)
    fsrc = jnp.broadcast_to(
        f2c.reshape(GB, 1, A, 6), (GB, HA, A, 6)).reshape(EP2, 6)
    fd = fdst - fsrc
    fd = fd - jnp.floor(fd)
    ones = jnp.ones((EP2, 1), f32)
    zero4 = jnp.zeros((EP2, 4), f32)
    fdh = jnp.concatenate(
        [fd[:, :3], ones, zero4, fd[:, 3:], ones, zero4], axis=1)
    fe = _sin2pi(dot(fdh, m2_ref[...]))

    lat9 = lat_ref[0]
    ip_cols = []
    for i in range(3):
        for j in range(3):
            s = (lat9[:, 3 * i + 0:3 * i + 1] * lat9[:, 3 * j + 0:3 * j + 1]
                 + lat9[:, 3 * i + 1:3 * i + 2] * lat9[:, 3 * j + 1:3 * j + 2]
                 + lat9[:, 3 * i + 2:3 * i + 3] * lat9[:, 3 * j + 2:3 * j + 3])
            ip_cols.append(s)
    latip = jnp.concatenate(ip_cols + [jnp.zeros((GB, 7), f32)], axis=1)

    inv_a = f32(1.0 / A)
    for l in range(NL):
        le = dot(latip, wlat_ref[l])
        le_n = jnp.broadcast_to(le[:, None, :], (GB, A, D)).reshape(NPB, D)
        pi = dot(nf, whi_ref[l]) + le_n + eb1_ref[l]
        pj = dot(nf, whj_ref[l])
        fdw = jnp.concatenate(
            [dot(fe[:, :64], wfd_ref[l]), dot(fe[:, 64:], wfd_ref[l])], axis=1)
        pi2 = jnp.concatenate([pi, pi], axis=1)
        hi_e = jnp.broadcast_to(
            pi2.reshape(GB, 1, A, D2), (GB, HA, A, D2)).reshape(EP2, D2)
        pj3 = pj.reshape(GB, A, D)
        pjp = jnp.concatenate([pj3[:, :HA, :], pj3[:, HA:, :]], axis=2)
        hj_e = jnp.broadcast_to(
            pjp[:, :, None, :], (GB, HA, A, D2)).reshape(EP2, D2)
        h = _silu2(hi_e + hj_e + fdw)
        ef = jnp.concatenate(
            [dot(h[:, :D], we2_ref[l]), dot(h[:, D:], we2_ref[l])], axis=1)
        ef = _silu2(ef + eb2_ref[l])
        agg = ((ef[:, :D] + ef[:, D:]).reshape(GB, HA, A, D)
               .sum(axis=1).reshape(NPB, D) * inv_a)
        h2 = _silu2(dot(nf, wn1a_ref[l]) + dot(agg, wn1b_ref[l]) + nb1_ref[l])
        nf = nf + _silu2(dot(h2, wn2_ref[l]) + nb2_ref[l])

    co = dot(nf, wc_ref[...])
    coord_ref[0] = co[:, :3]
    gf = nf.reshape(GB, A, D).sum(axis=1) * inv_a
    l9 = dot(gf, wl_ref[...])
    out_cols = []
    for i in range(3):
        for k in range(3):
            s = (l9[:, 3 * i + 0:3 * i + 1] * lat9[:, 0 + k:1 + k]
                 + l9[:, 3 * i + 1:3 * i + 2] * lat9[:, 3 + k:4 + k]
                 + l9[:, 3 * i + 2:3 * i + 3] * lat9[:, 6 + k:7 + k])
            out_cols.append(s)
    latout_ref[0] = jnp.concatenate(out_cols, axis=1)


def kernel(t, atom_types, frac_coords, lattices, num_atoms, node2graph,
           emb_table, W_latent, b_latent, edge_w1, edge_b1, edge_w2, edge_b2,
           node_w1, node_b1, node_w2, node_b2, W_coord, W_lattice):
    f32 = jnp.float32
    at_f = (atom_types.astype(f32) - 1.0).reshape(NB, NPB, 1)
    t_r = t.reshape(NB, GB, L)
    frac_r = frac_coords.reshape(NB, GB, A, 3)
    lat_r = lattices.reshape(NB, GB, 9)
    m2 = jnp.asarray(_angle_matrix())

    wla = W_latent[:, :D].T
    emb_pad = jnp.zeros((D, D), f32).at[:MAXA].set(emb_table) @ wla
    wlb = W_latent[:, D:].T
    bl = b_latent.reshape(1, D)
    e1t = jnp.swapaxes(edge_w1, 1, 2) * 0.5
    whi = e1t[:, :D]
    whj = e1t[:, D:2 * D]
    wlat = jnp.zeros((NL, 16, D), f32).at[:, :9].set(e1t[:, 2 * D:2 * D + 9])
    wfd = jnp.zeros((NL, 64, D), f32).at[:, :60].set(e1t[:, 2 * D + 9:])
    eb1 = edge_b1.reshape(NL, 1, D) * 0.5
    we2 = jnp.swapaxes(edge_w2, 1, 2) * 0.5
    eb2 = jnp.tile(edge_b2.reshape(NL, 1, D), (1, 1, 2)) * 0.5
    n1t = jnp.swapaxes(node_w1, 1, 2) * 0.5
    wn1a = n1t[:, :D]
    wn1b = n1t[:, D:]
    nb1 = node_b1.reshape(NL, 1, D) * 0.5
    wn2 = jnp.swapaxes(node_w2, 1, 2) * 0.5
    nb2 = node_b2.reshape(NL, 1, D) * 0.5
    wc = jnp.zeros((D, 8), f32).at[:, :3].set(W_coord.T)
    wl = jnp.zeros((D, 16), f32).at[:, :9].set(W_lattice.T)

    def blk(shape):
        return pl.BlockSpec(shape, lambda i: (i,) + (0,) * (len(shape) - 1))

    def const(shape):
        return pl.BlockSpec(shape, lambda i: (0,) * len(shape))

    coord_r, latout_r = pl.pallas_call(
        _fused_kernel,
        grid=(NB,),
        in_specs=[
            blk((1, NPB, 1)), blk((1, GB, L)), blk((1, GB, A, 3)), blk((1, GB, 9)),
            const((16, D)),
            const((D, D)), const((L, D)), const((1, D)),
            const((NL, D, D)), const((NL, D, D)), const((NL, 16, D)),
            const((NL, 64, D)), const((NL, 1, D)),
            const((NL, D, D)), const((NL, 1, D2)),
            const((NL, D, D)), const((NL, D, D)), const((NL, 1, D)),
            const((NL, D, D)), const((NL, 1, D)),
            const((D, 8)), const((D, 16)),
        ],
        out_specs=[blk((1, NPB, 3)), blk((1, GB, 9))],
        out_shape=[
            jax.ShapeDtypeStruct((NB, NPB, 3), f32),
            jax.ShapeDtypeStruct((NB, GB, 9), f32),
        ],
        compiler_params=pltpu.CompilerParams(
            dimension_semantics=("parallel",),
        ),
    )(at_f, t_r, frac_r, lat_r, m2, emb_pad, wlb, bl, whi, whj, wlat,
      wfd, eb1, we2, eb2, wn1a, wn1b, nb1, wn2, nb2, wc, wl)

    coord_out = coord_r.reshape(N, 3)
    lattice_out = latout_r.reshape(G, 3, 3)
    return lattice_out, coord_out


# speedup vs baseline: 1.4087x; 1.0119x over previous
import functools

import jax
import jax.numpy as jnp
import numpy as np
from jax.experimental import pallas as pl
from jax.experimental.pallas import tpu as pltpu

G, A, N, D, L, NL, MAXA, NF = 625, 16, 10000, 128, 256, 4, 100, 10
GB = 25
NB = G // GB
NPB = GB * A
EPB = GB * A * A
EP2 = EPB // 2
HA = A // 2
D2 = 2 * D
_TWO_PI = float(2.0 * np.pi)


def _angle_matrix():
    m = np.zeros((16, 128), np.float32)
    for half in range(2):
        r0, c0 = 8 * half, 64 * half
        for c in range(3):
            for f in range(NF):
                m[r0 + c, c0 + c * NF + f] = float(f)
                m[r0 + c, c0 + 30 + c * NF + f] = float(f)
        m[r0 + 3, c0 + 30:c0 + 60] = 0.25
    return m


_SIN_C = (6.283055827840459, -41.33121607498482, 81.36684415521783,
          -74.47817705034134, 32.78174054516373)


def _sin2pi(m):
    r = m - jnp.round(m)
    t = r * r
    p = jnp.float32(_SIN_C[4])
    for c in _SIN_C[3::-1]:
        p = p * t + jnp.float32(c)
    return r * p


def _silu2(u):
    return u + u * jnp.tanh(u)


def _fused_kernel(at_ref, t_ref, frac_ref, lat_ref,
                  m2_ref,
                  emb_ref, wlb_ref, bl_ref,
                  whi_ref, whj_ref, wlat_ref, wfd_ref, eb1_ref,
                  we2_ref, eb2_ref,
                  wn1a_ref, wn1b_ref, nb1_ref, wn2_ref, nb2_ref,
                  wc_ref, wl_ref,
                  coord_ref, latout_ref):
    f32 = jnp.float32
    dot = functools.partial(jnp.dot, preferred_element_type=f32)

    at = at_ref[0]
    lane = jax.lax.broadcasted_iota(jnp.int32, (NPB, D), 1).astype(f32)
    onehot = (lane == at).astype(f32)
    tb = t_ref[0]
    tw = dot(tb, wlb_ref[...]) + bl_ref[...]
    tw_n = jnp.broadcast_to(tw[:, None, :], (GB, A, D)).reshape(NPB, D)
    nf = dot(onehot, emb_ref[...]) + tw_n

    f3 = frac_ref[0]
    f2 = f3.reshape(NPB, 3)
    f6 = jnp.concatenate([f3[:, :HA, :], f3[:, HA:, :]], axis=2)
    fdst = jnp.broadcast_to(f6[:, :, None, :], (GB, HA, A, 6)).reshape(EP2, 6)
    f2c = jnp.concatenate([f2, f2], axis=1)
    fsrc = jnp.broadcast_to(
        f2c.reshape(GB, 1, A, 6), (GB, HA, A, 6)).reshape(EP2, 6)
    fd = fdst - fsrc
    fd = fd - jnp.floor(fd)
    ones = jnp.ones((EP2, 1), f32)
    zero4 = jnp.zeros((EP2, 4), f32)
    fdh = jnp.concatenate(
        [fd[:, :3], ones, zero4, fd[:, 3:], ones, zero4], axis=1)
    fe = _sin2pi(dot(fdh, m2_ref[...]))

    lat9 = lat_ref[0]
    ip_cols = []
    for i in range(3):
        for j in range(3):
            s = (lat9[:, 3 * i + 0:3 * i + 1] * lat9[:, 3 * j + 0:3 * j + 1]
                 + lat9[:, 3 * i + 1:3 * i + 2] * lat9[:, 3 * j + 1:3 * j + 2]
                 + lat9[:, 3 * i + 2:3 * i + 3] * lat9[:, 3 * j + 2:3 * j + 3])
            ip_cols.append(s)
    latip = jnp.concatenate(ip_cols + [jnp.zeros((GB, 7), f32)], axis=1)

    for l in range(NL):
        le = dot(latip, wlat_ref[l])
        le_n = jnp.broadcast_to(le[:, None, :], (GB, A, D)).reshape(NPB, D)
        pi = dot(nf, whi_ref[l]) + le_n + eb1_ref[l]
        pj = dot(nf, whj_ref[l])
        fdw = jnp.concatenate(
            [dot(fe[:, :64], wfd_ref[l]), dot(fe[:, 64:], wfd_ref[l])], axis=1)
        pi2 = jnp.concatenate([pi, pi], axis=1)
        hi_e = jnp.broadcast_to(
            pi2.reshape(GB, 1, A, D2), (GB, HA, A, D2)).reshape(EP2, D2)
        pj3 = pj.reshape(GB, A, D)
        pjp = jnp.concatenate([pj3[:, :HA, :], pj3[:, HA:, :]], axis=2)
        hj_e = jnp.broadcast_to(
            pjp[:, :, None, :], (GB, HA, A, D2)).reshape(EP2, D2)
        h = _silu2(hi_e + hj_e + fdw)
        ef = jnp.concatenate(
            [dot(h[:, :D], we2_ref[l]), dot(h[:, D:], we2_ref[l])], axis=1)
        ef = _silu2(ef + eb2_ref[l])
        agg = ((ef[:, :D] + ef[:, D:]).reshape(GB, HA, A, D)
               .sum(axis=1).reshape(NPB, D))
        h2 = _silu2(dot(nf, wn1a_ref[l]) + dot(agg, wn1b_ref[l]) + nb1_ref[l])
        nf = nf + _silu2(dot(h2, wn2_ref[l]) + nb2_ref[l])

    co = dot(nf, wc_ref[...])
    coord_ref[0] = co[:, :3]
    gf = nf.reshape(GB, A, D).sum(axis=1)
    l9 = dot(gf, wl_ref[...])
    out_cols = []
    for i in range(3):
        for k in range(3):
            s = (l9[:, 3 * i + 0:3 * i + 1] * lat9[:, 0 + k:1 + k]
                 + l9[:, 3 * i + 1:3 * i + 2] * lat9[:, 3 + k:4 + k]
                 + l9[:, 3 * i + 2:3 * i + 3] * lat9[:, 6 + k:7 + k])
            out_cols.append(s)
    latout_ref[0] = jnp.concatenate(out_cols, axis=1)


def kernel(t, atom_types, frac_coords, lattices, num_atoms, node2graph,
           emb_table, W_latent, b_latent, edge_w1, edge_b1, edge_w2, edge_b2,
           node_w1, node_b1, node_w2, node_b2, W_coord, W_lattice):
    f32 = jnp.float32
    at_f = (atom_types.astype(f32) - 1.0).reshape(NB, NPB, 1)
    t_r = t.reshape(NB, GB, L)
    frac_r = frac_coords.reshape(NB, GB, A, 3)
    lat_r = lattices.reshape(NB, GB, 9)
    m2 = jnp.asarray(_angle_matrix())

    wla = W_latent[:, :D].T
    emb_pad = jnp.zeros((D, D), f32).at[:MAXA].set(emb_table) @ wla
    wlb = W_latent[:, D:].T
    bl = b_latent.reshape(1, D)
    e1t = jnp.swapaxes(edge_w1, 1, 2) * 0.5
    whi = e1t[:, :D]
    whj = e1t[:, D:2 * D]
    wlat = jnp.zeros((NL, 16, D), f32).at[:, :9].set(e1t[:, 2 * D:2 * D + 9])
    wfd = jnp.zeros((NL, 64, D), f32).at[:, :60].set(e1t[:, 2 * D + 9:])
    eb1 = edge_b1.reshape(NL, 1, D) * 0.5
    we2 = jnp.swapaxes(edge_w2, 1, 2) * 0.5
    eb2 = jnp.tile(edge_b2.reshape(NL, 1, D), (1, 1, 2)) * 0.5
    n1t = jnp.swapaxes(node_w1, 1, 2) * 0.5
    wn1a = n1t[:, :D]
    wn1b = n1t[:, D:] * (1.0 / A)
    nb1 = node_b1.reshape(NL, 1, D) * 0.5
    wn2 = jnp.swapaxes(node_w2, 1, 2) * 0.5
    nb2 = node_b2.reshape(NL, 1, D) * 0.5
    wc = jnp.zeros((D, 8), f32).at[:, :3].set(W_coord.T)
    wl = jnp.zeros((D, 16), f32).at[:, :9].set(W_lattice.T * (1.0 / A))

    def blk(shape):
        return pl.BlockSpec(shape, lambda i: (i,) + (0,) * (len(shape) - 1))

    def const(shape):
        return pl.BlockSpec(shape, lambda i: (0,) * len(shape))

    coord_r, latout_r = pl.pallas_call(
        _fused_kernel,
        grid=(NB,),
        in_specs=[
            blk((1, NPB, 1)), blk((1, GB, L)), blk((1, GB, A, 3)), blk((1, GB, 9)),
            const((16, D)),
            const((D, D)), const((L, D)), const((1, D)),
            const((NL, D, D)), const((NL, D, D)), const((NL, 16, D)),
            const((NL, 64, D)), const((NL, 1, D)),
            const((NL, D, D)), const((NL, 1, D2)),
            const((NL, D, D)), const((NL, D, D)), const((NL, 1, D)),
            const((NL, D, D)), const((NL, 1, D)),
            const((D, 8)), const((D, 16)),
        ],
        out_specs=[blk((1, NPB, 3)), blk((1, GB, 9))],
        out_shape=[
            jax.ShapeDtypeStruct((NB, NPB, 3), f32),
            jax.ShapeDtypeStruct((NB, GB, 9), f32),
        ],
        compiler_params=pltpu.CompilerParams(
            dimension_semantics=("parallel",),
        ),
    )(at_f, t_r, frac_r, lat_r, m2, emb_pad, wlb, bl, whi, whj, wlat,
      wfd, eb1, we2, eb2, wn1a, wn1b, nb1, wn2, nb2, wc, wl)

    coord_out = coord_r.reshape(N, 3)
    lattice_out = latout_r.reshape(G, 3, 3)
    return lattice_out, coord_out
